# baseline (speedup 1.0000x reference)
import jax
import jax.numpy as jnp
from jax import lax
from jax.experimental import pallas as pl
from jax.experimental.pallas import tpu as pltpu
from jax.experimental.pallas import tpu_sc as plsc

B, T, D = 16, 512, 256
L = 2048
MAX_DUR = 8

NC, NS = 2, 16
HALF = L // 2
CHUNK = 32
NCHUNK = HALF // CHUNK
NBUF = 4
SROWS = CHUNK + 8
ZROW = SROWS
LANES = 16
LPAD = L + 2 * LANES


def _body(enc_hbm, dur_hbm, out_hbm, dur_v, idx_v, st0, st1, st2, st3,
          ob0, ob1, ob2, ob3, ssem0, ssem1, ssem2, ssem3,
          osem0, osem1, osem2, osem3):
  cid = lax.axis_index("c")
  sid = lax.axis_index("s")
  wid = sid * NC + cid
  b = wid // 2
  half = wid % 2
  row0 = b * T

  pltpu.sync_copy(dur_hbm.at[b], dur_v)

  zf = jnp.zeros((LANES,), jnp.float32)
  stbufs = (st0, st1, st2, st3)
  obufs = (ob0, ob1, ob2, ob3)
  ssems = (ssem0, ssem1, ssem2, ssem3)
  osems = (osem0, osem1, osem2, osem3)

  for stb in stbufs:
    for j in range(D // LANES):
      stb[ZROW, pl.ds(j * LANES, LANES)] = zf

  zi = jnp.full((LANES,), row0, jnp.int32)
  for j in range(LPAD // LANES):
    idx_v[pl.ds(j * LANES, LANES)] = zi

  t_iota = lax.iota(jnp.int32, LANES)
  carry = jnp.int32(0)
  for i in range(T // LANES):
    v = dur_v[pl.ds(i * LANES, LANES)]
    inc = plsc.cumsum(v)
    start = inc - v + carry
    carry = carry + inc[LANES - 1]
    val = t_iota + (row0 + i * LANES)
    for k in range(MAX_DUR - 1):
      pos = start + k
      mask = (v > k) & (pos < L)
      plsc.store_scatter(idx_v, [pos], val, mask=mask)
  total = carry

  def _off(ci):
    return (2 * ci + half) * CHUNK

  def _stage_base(ci):
    off = _off(jnp.minimum(ci, NCHUNK - 1))
    t0g = idx_v[pl.ds(off, LANES)][0]
    sbl = jnp.clip(t0g - row0, 0, T - SROWS)
    return pl.multiple_of(jnp.bitwise_and(sbl, -8), 8)

  def _stage_issue(sb, stbuf, sem):
    pltpu.async_copy(
        enc_hbm.at[pl.ds(row0 + sb, SROWS), :], stbuf.at[pl.ds(0, SROWS), :], sem
    )

  sb_list = []
  for s in range(NBUF):
    sb = _stage_base(jnp.int32(s))
    _stage_issue(sb, stbufs[s], ssems[s])
    sb_list.append(sb)

  def _outer(oi, sbs):
    new_sbs = []
    for s in range(NBUF):
      ci = oi * NBUF + s
      off = _off(ci)
      stb = stbufs[s]
      ob = obufs[s]
      sb = sbs[s]

      pltpu.make_async_copy(
          enc_hbm.at[pl.ds(row0, SROWS), :], stb.at[pl.ds(0, SROWS), :], ssems[s]
      ).wait()

      @pl.when(oi > 0)
      def _drain():
        pltpu.make_async_copy(ob, out_hbm.at[pl.ds(b * L + off, CHUNK), :], osems[s]).wait()

      base = row0 + sb

      @plsc.parallel_loop(0, CHUNK, unroll=8)
      def _expand(fr):
        lv = idx_v[pl.ds(off + fr, LANES)][0] - base
        lt = jnp.where(off + fr >= total, ZROW, jnp.clip(lv, 0, SROWS - 1))
        vals = [stb[lt, pl.ds(j * LANES, LANES)] for j in range(D // LANES)]
        for j in range(D // LANES):
          ob[fr, pl.ds(j * LANES, LANES)] = vals[j]

      sb_next = _stage_base(ci + NBUF)

      @pl.when(oi < NCHUNK // NBUF - 1)
      def _refill():
        _stage_issue(sb_next, stb, ssems[s])

      new_sbs.append(sb_next)

      pltpu.async_copy(ob, out_hbm.at[pl.ds(b * L + off, CHUNK), :], osems[s])
    return tuple(new_sbs)

  lax.fori_loop(0, NCHUNK // NBUF, _outer, tuple(sb_list))

  for s in range(NBUF):
    pltpu.make_async_copy(obufs[s], out_hbm.at[pl.ds(b * L, CHUNK), :], osems[s]).wait()


@jax.jit
def kernel(encoder_output, durations):
  enc_flat = encoder_output.reshape(B * T, D)
  dur32 = durations.astype(jnp.int32)
  mesh = plsc.VectorSubcoreMesh(
      core_axis_name="c", subcore_axis_name="s", num_cores=NC, num_subcores=NS
  )
  run = pl.kernel(
      _body,
      out_type=jax.ShapeDtypeStruct((B * L, D), jnp.float32),
      mesh=mesh,
      scratch_types=(
          [pltpu.VMEM((T,), jnp.int32), pltpu.VMEM((LPAD,), jnp.int32)]
          + [pltpu.VMEM((SROWS + 1, D), jnp.float32)] * NBUF
          + [pltpu.VMEM((CHUNK, D), jnp.float32)] * NBUF
          + [pltpu.SemaphoreType.DMA] * (2 * NBUF)
      ),
      compiler_params=pltpu.CompilerParams(needs_layout_passes=False),
  )
  out = run(enc_flat, dur32)
  return out.reshape(B, L, D)


# speedup vs baseline: 1.0787x; 1.0787x over previous
import jax
import jax.numpy as jnp
from jax import lax
from jax.experimental import pallas as pl
from jax.experimental.pallas import tpu as pltpu
from jax.experimental.pallas import tpu_sc as plsc

B, T, D = 16, 512, 256
L = 2048
MAX_DUR = 8

NC, NS = 2, 16
HALF = L // 2
CHUNK = 64
NCHUNK = HALF // CHUNK
NBUF = 2
SROWS = CHUNK + 8
ZROW = SROWS
LANES = 16
LPAD = L + 2 * LANES


def _body(enc_hbm, dur_hbm, out_hbm, dur_v, idx_v, *bufs_and_sems):
  stbufs = bufs_and_sems[:NBUF]
  obufs = bufs_and_sems[NBUF:2 * NBUF]
  ssems = bufs_and_sems[2 * NBUF:3 * NBUF]
  osems = bufs_and_sems[3 * NBUF:4 * NBUF]
  cid = lax.axis_index("c")
  sid = lax.axis_index("s")
  wid = sid * NC + cid
  b = wid // 2
  half = wid % 2
  row0 = b * T

  pltpu.sync_copy(dur_hbm.at[b], dur_v)

  zf = jnp.zeros((LANES,), jnp.float32)

  for stb in stbufs:
    for j in range(D // LANES):
      stb[ZROW, pl.ds(j * LANES, LANES)] = zf

  zi = jnp.full((LANES,), row0, jnp.int32)
  for j in range(LPAD // LANES):
    idx_v[pl.ds(j * LANES, LANES)] = zi

  t_iota = lax.iota(jnp.int32, LANES)
  carry = jnp.int32(0)
  for i in range(T // LANES):
    v = dur_v[pl.ds(i * LANES, LANES)]
    inc = plsc.cumsum(v)
    start = inc - v + carry
    carry = carry + inc[LANES - 1]
    val = t_iota + (row0 + i * LANES)
    for k in range(MAX_DUR - 1):
      pos = start + k
      mask = (v > k) & (pos < L)
      plsc.store_scatter(idx_v, [pos], val, mask=mask)
  total = carry

  def _off(ci):
    return (2 * ci + half) * CHUNK

  def _stage_base(ci):
    off = _off(jnp.minimum(ci, NCHUNK - 1))
    t0g = idx_v[pl.ds(off, LANES)][0]
    sbl = jnp.clip(t0g - row0, 0, T - SROWS)
    return pl.multiple_of(jnp.bitwise_and(sbl, -8), 8)

  def _stage_issue(sb, stbuf, sem):
    pltpu.async_copy(
        enc_hbm.at[pl.ds(row0 + sb, SROWS), :], stbuf.at[pl.ds(0, SROWS), :], sem
    )

  sb_list = []
  for s in range(NBUF):
    sb = _stage_base(jnp.int32(s))
    _stage_issue(sb, stbufs[s], ssems[s])
    sb_list.append(sb)

  def _outer(oi, sbs):
    new_sbs = []
    for s in range(NBUF):
      ci = oi * NBUF + s
      off = _off(ci)
      stb = stbufs[s]
      ob = obufs[s]
      sb = sbs[s]

      pltpu.make_async_copy(
          enc_hbm.at[pl.ds(row0, SROWS), :], stb.at[pl.ds(0, SROWS), :], ssems[s]
      ).wait()

      @pl.when(oi > 0)
      def _drain():
        pltpu.make_async_copy(ob, out_hbm.at[pl.ds(b * L + off, CHUNK), :], osems[s]).wait()

      base = row0 + sb

      @plsc.parallel_loop(0, CHUNK, unroll=4)
      def _expand(fr):
        lv = idx_v[pl.ds(off + fr, LANES)][0] - base
        lt = jnp.where(off + fr >= total, ZROW, jnp.clip(lv, 0, SROWS - 1))
        vals = [stb[lt, pl.ds(j * LANES, LANES)] for j in range(D // LANES)]
        for j in range(D // LANES):
          ob[fr, pl.ds(j * LANES, LANES)] = vals[j]

      sb_next = _stage_base(ci + NBUF)

      @pl.when(oi < NCHUNK // NBUF - 1)
      def _refill():
        _stage_issue(sb_next, stb, ssems[s])

      new_sbs.append(sb_next)

      pltpu.async_copy(ob, out_hbm.at[pl.ds(b * L + off, CHUNK), :], osems[s])
    return tuple(new_sbs)

  lax.fori_loop(0, NCHUNK // NBUF, _outer, tuple(sb_list))

  for s in range(NBUF):
    pltpu.make_async_copy(obufs[s], out_hbm.at[pl.ds(b * L, CHUNK), :], osems[s]).wait()


@jax.jit
def kernel(encoder_output, durations):
  enc_flat = encoder_output.reshape(B * T, D)
  dur32 = durations.astype(jnp.int32)
  mesh = plsc.VectorSubcoreMesh(
      core_axis_name="c", subcore_axis_name="s", num_cores=NC, num_subcores=NS
  )
  run = pl.kernel(
      _body,
      out_type=jax.ShapeDtypeStruct((B * L, D), jnp.float32),
      mesh=mesh,
      scratch_types=(
          [pltpu.VMEM((T,), jnp.int32), pltpu.VMEM((LPAD,), jnp.int32)]
          + [pltpu.VMEM((SROWS + 1, D), jnp.float32)] * NBUF
          + [pltpu.VMEM((CHUNK, D), jnp.float32)] * NBUF
          + [pltpu.SemaphoreType.DMA] * (2 * NBUF)
      ),
      compiler_params=pltpu.CompilerParams(needs_layout_passes=False),
  )
  out = run(enc_flat, dur32)
  return out.reshape(B, L, D)
